# baseline (speedup 1.0000x reference)
import functools

import jax
import jax.numpy as jnp
from jax import lax
from jax.experimental import pallas as pl
from jax.experimental.pallas import tpu as pltpu
from jax.experimental.pallas import tpu_sc as plsc

N_NODES = 10000
N_EDGES = 320000
NC = 2
NS = 16
NW = NC * NS
N_PAD = 10240
RPT = N_PAD // NS
K = 32
CHUNKS = 314
EPW = CHUNKS * K
E_PAD = EPW * NW

_SC_PARAMS = dict(
    compiler_params=pltpu.CompilerParams(needs_layout_passes=False),
)


def _mesh():
    return plsc.VectorSubcoreMesh(core_axis_name="c", subcore_axis_name="s")


@functools.lru_cache(maxsize=None)
def _deg_kernel():
    def body(ew_hbm, col_hbm, zero_hbm, out_hbm, colv, ewv, acc, sem):
        c = lax.axis_index("c")
        s = lax.axis_index("s")
        w = c * NS + s
        pltpu.sync_copy(zero_hbm.at[pl.ds(s * RPT, RPT)],
                        acc.at[pl.ds(s * RPT, RPT)])
        pltpu.sync_copy(col_hbm.at[w], colv)
        pltpu.sync_copy(ew_hbm.at[w], ewv)
        plsc.subcore_barrier()

        def chunk(i, carry):
            pltpu.sync_copy(ewv.at[i], acc.at[colv.at[i]], add=True)
            return carry

        lax.fori_loop(0, CHUNKS, chunk, 0)
        plsc.subcore_barrier()
        pltpu.sync_copy(acc.at[pl.ds(s * RPT, RPT)],
                        out_hbm.at[c, pl.ds(s * RPT, RPT)])

    return pl.kernel(
        body,
        mesh=_mesh(),
        out_type=jax.ShapeDtypeStruct((NC, N_PAD), jnp.float32),
        scratch_types=[
            pltpu.VMEM((CHUNKS, K), jnp.int32),
            pltpu.VMEM((CHUNKS, K), jnp.float32),
            pltpu.VMEM_SHARED((N_PAD,), jnp.float32),
            pltpu.SemaphoreType.DMA,
        ],
        **_SC_PARAMS,
    )


@functools.lru_cache(maxsize=None)
def _make_prop(d):
    def scale_rows(rows, ewv, _):
        for j16 in range(K // 16):
            ew16 = ewv[pl.ds(j16 * 16, 16)]
            for jj in range(16):
                j = j16 * 16 + jj
                wb = ew16.at[jnp.full((16,), jj, jnp.int32)].get(
                    mode="promise_in_bounds")
                for cc in range(d // 16):
                    rows[j, pl.ds(cc * 16, 16)] = (
                        rows[j, pl.ds(cc * 16, 16)] * wb)

    def body(g_hbm, row_hbm, col_hbm, ew_hbm, zero_hbm, out_hbm,
             row0, row1, col0, col1, ew0, ew1, rows0, rows1, acc,
             gsem0, gsem1, isem0, isem1, csem0, csem1, ssem0, ssem1):
        c = lax.axis_index("c")
        s = lax.axis_index("s")
        w = c * NS + s
        pltpu.sync_copy(zero_hbm.at[pl.ds(s * RPT, RPT)],
                        acc.at[pl.ds(s * RPT, RPT)])
        plsc.subcore_barrier()

        rowb = (row0, row1)
        colb = (col0, col1)
        ewb = (ew0, ew1)
        rowsb = (rows0, rows1)
        gsem = (gsem0, gsem1)
        isem = (isem0, isem1)
        csem = (csem0, csem1)
        ssem = (ssem0, ssem1)

        def issue_rowew(ci, j):
            pltpu.async_copy(row_hbm.at[w, ci], rowb[j], isem[j])
            pltpu.async_copy(ew_hbm.at[w, ci], ewb[j], isem[j])

        def wait_rowew(ci, j):
            pltpu.make_async_copy(row_hbm.at[w, ci], rowb[j], isem[j]).wait()
            pltpu.make_async_copy(ew_hbm.at[w, ci], ewb[j], isem[j]).wait()

        def issue_col(ci, j):
            pltpu.async_copy(col_hbm.at[w, ci], colb[j], csem[j])

        def wait_col(ci, j):
            pltpu.make_async_copy(col_hbm.at[w, ci], colb[j], csem[j]).wait()

        def issue_gather(j):
            pltpu.async_copy(g_hbm.at[rowb[j]], rowsb[j], gsem[j])

        def wait_gather(j):
            pltpu.make_async_copy(g_hbm.at[rowb[j]], rowsb[j], gsem[j]).wait()

        def start_scatter(j):
            pltpu.async_copy(rowsb[j], acc.at[colb[j]], ssem[j], add=True)

        def wait_scatter(j):
            pltpu.make_async_copy(rowsb[j], acc.at[colb[j]], ssem[j]).wait()

        issue_rowew(0, 0)
        wait_rowew(0, 0)
        issue_gather(0)
        issue_col(0, 0)
        issue_rowew(1, 1)

        def step(ci, j, first):
            @pl.when(ci + 1 < CHUNKS)
            def _():
                wait_rowew(ci + 1, 1 - j)
                if not first:
                    wait_scatter(1 - j)
                issue_gather(1 - j)
                issue_col(ci + 1, 1 - j)

            wait_gather(j)
            scale_rows(rowsb[j], ewb[j], None)
            wait_col(ci, j)
            start_scatter(j)

            @pl.when(ci + 2 < CHUNKS)
            def _():
                issue_rowew(ci + 2, j)

        def first_pair(i, carry):
            step(0, 0, True)
            step(1, 1, False)
            return carry

        def pair(i, carry):
            ci = 2 * i
            step(ci, 0, False)
            step(ci + 1, 1, False)
            return carry

        first_pair(0, 0)
        lax.fori_loop(1, CHUNKS // 2, pair, 0)
        wait_scatter(0)
        wait_scatter(1)
        plsc.subcore_barrier()
        pltpu.sync_copy(acc.at[pl.ds(s * RPT, RPT)],
                        out_hbm.at[c, pl.ds(s * RPT, RPT)])

    return pl.kernel(
        body,
        mesh=_mesh(),
        out_type=jax.ShapeDtypeStruct((NC, N_PAD, d), jnp.float32),
        scratch_types=[
            pltpu.VMEM((K,), jnp.int32),
            pltpu.VMEM((K,), jnp.int32),
            pltpu.VMEM((K,), jnp.int32),
            pltpu.VMEM((K,), jnp.int32),
            pltpu.VMEM((K,), jnp.float32),
            pltpu.VMEM((K,), jnp.float32),
            pltpu.VMEM((K, d), jnp.float32),
            pltpu.VMEM((K, d), jnp.float32),
            pltpu.VMEM_SHARED((N_PAD, d), jnp.float32),
            pltpu.SemaphoreType.DMA,
            pltpu.SemaphoreType.DMA,
            pltpu.SemaphoreType.DMA,
            pltpu.SemaphoreType.DMA,
            pltpu.SemaphoreType.DMA,
            pltpu.SemaphoreType.DMA,
            pltpu.SemaphoreType.DMA,
            pltpu.SemaphoreType.DMA,
        ],
        **_SC_PARAMS,
    )


_BR = 1024
_GRID = (N_PAD // _BR,)


def _tc_first(x_ref, w_ref, dinvb_ref, o_ref):
    o_ref[...] = dinvb_ref[...] * jnp.dot(
        x_ref[...], w_ref[...], preferred_element_type=jnp.float32)


def _tc_mid(acc_ref, g_ref, dinvb_ref, w_ref, b_ref, o_ref):
    dinvb = dinvb_ref[...]
    z = dinvb[:, : g_ref.shape[1]] * (acc_ref[0] + acc_ref[1] + g_ref[...]) \
        + b_ref[...][0:1, :]
    h = jnp.maximum(z, 0.0)
    o_ref[...] = dinvb[:, : w_ref.shape[1]] * jnp.dot(
        h, w_ref[...], preferred_element_type=jnp.float32)


def _tc_last(acc_ref, g_ref, dinvb_ref, b_ref, o_ref):
    o_ref[...] = dinvb_ref[...][:, :2] * (
        acc_ref[0][:, :2] + acc_ref[1][:, :2] + g_ref[...][:, :2]
    ) + b_ref[...][0:1, :]


def _row_spec(d):
    return pl.BlockSpec((_BR, d), lambda i: (i, 0))


def _acc_spec(d):
    return pl.BlockSpec((NC, _BR, d), lambda i: (0, i, 0))


def _full_spec(a, b):
    return pl.BlockSpec((a, b), lambda i: (0, 0))


def _tc_first_call(xp, W1, dinvb):
    return pl.pallas_call(
        _tc_first,
        grid=_GRID,
        in_specs=[_row_spec(128), _full_spec(128, 128), _row_spec(128)],
        out_specs=_row_spec(128),
        out_shape=jax.ShapeDtypeStruct((N_PAD, 128), jnp.float32),
    )(xp, W1, dinvb)


def _tc_mid_call(acc, g, dinvb, W, b2d, dout):
    din = g.shape[1]
    return pl.pallas_call(
        _tc_mid,
        grid=_GRID,
        in_specs=[_acc_spec(din), _row_spec(din), _row_spec(128),
                  _full_spec(128, dout), _full_spec(8, din)],
        out_specs=_row_spec(dout),
        out_shape=jax.ShapeDtypeStruct((N_PAD, dout), jnp.float32),
    )(acc, g, dinvb, W, b2d)


def _tc_last_call(acc, g, dinvb, b2d):
    return pl.pallas_call(
        _tc_last,
        grid=_GRID,
        in_specs=[_acc_spec(128), _row_spec(128), _row_spec(128),
                  _full_spec(8, 2)],
        out_specs=_row_spec(2),
        out_shape=jax.ShapeDtypeStruct((N_PAD, 2), jnp.float32),
    )(acc, g, dinvb, b2d)


def kernel(x, edge_index, edge_attr, W1, b1, W2, b2, W3, b3):
    row = edge_index[0].astype(jnp.int32)
    col = edge_index[1].astype(jnp.int32)
    ew = edge_attr.astype(jnp.float32)

    pe = E_PAD - N_EDGES
    pad_idx = jnp.arange(pe, dtype=jnp.int32) % N_PAD
    rowp = jnp.concatenate([row, pad_idx]).reshape(NW, CHUNKS, K)
    colp = jnp.concatenate([col, pad_idx]).reshape(NW, CHUNKS, K)
    ewp = jnp.pad(ew, (0, pe)).reshape(NW, CHUNKS, K)
    xp = jnp.pad(x, ((0, N_PAD - N_NODES), (0, 0)))

    zero1 = jnp.zeros((N_PAD,), jnp.float32)
    zero128 = jnp.zeros((N_PAD, 128), jnp.float32)

    deg_parts = _deg_kernel()(ewp, colp, zero1)
    deg = deg_parts[0] + deg_parts[1] + 1.0
    dinv = jnp.where(deg > 0, lax.rsqrt(deg), 0.0)
    dinvb = jnp.broadcast_to(dinv[:, None], (N_PAD, 128))

    b1b = jnp.broadcast_to(b1[None, :], (8, 128))
    b2b = jnp.broadcast_to(b2[None, :], (8, 128))
    b3b = jnp.broadcast_to(b3[None, :], (8, 2))
    W3p = jnp.zeros((128, 128), jnp.float32).at[:, :2].set(W3)

    g1 = _tc_first_call(xp, W1, dinvb)
    acc1 = _make_prop(128)(g1, rowp, colp, ewp, zero128)
    g2 = _tc_mid_call(acc1, g1, dinvb, W2, b1b, 128)
    acc2 = _make_prop(128)(g2, rowp, colp, ewp, zero128)
    g3 = _tc_mid_call(acc2, g2, dinvb, W3p, b2b, 128)
    acc3 = _make_prop(128)(g3, rowp, colp, ewp, zero128)
    out = _tc_last_call(acc3, g3, dinvb, b3b)
    return out[:N_NODES]


# speedup vs baseline: 1.4410x; 1.4410x over previous
import functools

import jax
import jax.numpy as jnp
from jax import lax
from jax.experimental import pallas as pl
from jax.experimental.pallas import tpu as pltpu
from jax.experimental.pallas import tpu_sc as plsc

N_NODES = 10000
N_EDGES = 320000
NC = 2
NS = 16
NW = NC * NS
N_PAD = 10240
RPT = N_PAD // NS
K = 64
CHUNKS = 158
EPW = CHUNKS * K
E_PAD = EPW * NW

_SC_PARAMS = dict(
    compiler_params=pltpu.CompilerParams(needs_layout_passes=False),
)


def _mesh():
    return plsc.VectorSubcoreMesh(core_axis_name="c", subcore_axis_name="s")


@functools.lru_cache(maxsize=None)
def _deg_kernel():
    def body(ew_hbm, col_hbm, zero_hbm, out_hbm, colv, ewv, acc, sem):
        c = lax.axis_index("c")
        s = lax.axis_index("s")
        w = c * NS + s
        pltpu.sync_copy(zero_hbm.at[pl.ds(s * RPT, RPT)],
                        acc.at[pl.ds(s * RPT, RPT)])
        pltpu.sync_copy(col_hbm.at[w], colv)
        pltpu.sync_copy(ew_hbm.at[w], ewv)
        plsc.subcore_barrier()

        def chunk(i, carry):
            pltpu.sync_copy(ewv.at[i], acc.at[colv.at[i]], add=True)
            return carry

        lax.fori_loop(0, CHUNKS, chunk, 0)
        plsc.subcore_barrier()
        pltpu.sync_copy(acc.at[pl.ds(s * RPT, RPT)],
                        out_hbm.at[c, pl.ds(s * RPT, RPT)])

    return pl.kernel(
        body,
        mesh=_mesh(),
        out_type=jax.ShapeDtypeStruct((NC, N_PAD), jnp.float32),
        scratch_types=[
            pltpu.VMEM((CHUNKS, K), jnp.int32),
            pltpu.VMEM((CHUNKS, K), jnp.float32),
            pltpu.VMEM_SHARED((N_PAD,), jnp.float32),
            pltpu.SemaphoreType.DMA,
        ],
        **_SC_PARAMS,
    )


@functools.lru_cache(maxsize=None)
def _make_prop(d):
    def scale_rows(rows, ewv, _):
        for j16 in range(K // 16):
            ew16 = ewv[pl.ds(j16 * 16, 16)]
            for jj in range(16):
                j = j16 * 16 + jj
                wb = ew16.at[jnp.full((16,), jj, jnp.int32)].get(
                    mode="promise_in_bounds")
                for cc in range(d // 16):
                    rows[j, pl.ds(cc * 16, 16)] = (
                        rows[j, pl.ds(cc * 16, 16)] * wb)

    def body(g_hbm, row_hbm, col_hbm, ew_hbm, zero_hbm, out_hbm,
             row0, row1, col0, col1, ew0, ew1, rows0, rows1, acc,
             gsem0, gsem1, isem0, isem1, csem0, csem1, ssem0, ssem1):
        c = lax.axis_index("c")
        s = lax.axis_index("s")
        w = c * NS + s
        pltpu.sync_copy(zero_hbm.at[pl.ds(s * RPT, RPT)],
                        acc.at[pl.ds(s * RPT, RPT)])
        plsc.subcore_barrier()

        rowb = (row0, row1)
        colb = (col0, col1)
        ewb = (ew0, ew1)
        rowsb = (rows0, rows1)
        gsem = (gsem0, gsem1)
        isem = (isem0, isem1)
        csem = (csem0, csem1)
        ssem = (ssem0, ssem1)

        def issue_rowew(ci, j):
            pltpu.async_copy(row_hbm.at[w, ci], rowb[j], isem[j])
            pltpu.async_copy(ew_hbm.at[w, ci], ewb[j], isem[j])

        def wait_rowew(ci, j):
            pltpu.make_async_copy(row_hbm.at[w, ci], rowb[j], isem[j]).wait()
            pltpu.make_async_copy(ew_hbm.at[w, ci], ewb[j], isem[j]).wait()

        def issue_col(ci, j):
            pltpu.async_copy(col_hbm.at[w, ci], colb[j], csem[j])

        def wait_col(ci, j):
            pltpu.make_async_copy(col_hbm.at[w, ci], colb[j], csem[j]).wait()

        def issue_gather(j):
            pltpu.async_copy(g_hbm.at[rowb[j]], rowsb[j], gsem[j])

        def wait_gather(j):
            pltpu.make_async_copy(g_hbm.at[rowb[j]], rowsb[j], gsem[j]).wait()

        def start_scatter(j):
            pltpu.async_copy(rowsb[j], acc.at[colb[j]], ssem[j], add=True)

        def wait_scatter(j):
            pltpu.make_async_copy(rowsb[j], acc.at[colb[j]], ssem[j]).wait()

        issue_rowew(0, 0)
        wait_rowew(0, 0)
        issue_gather(0)
        issue_col(0, 0)
        issue_rowew(1, 1)

        def step(ci, j, first):
            @pl.when(ci + 1 < CHUNKS)
            def _():
                wait_rowew(ci + 1, 1 - j)
                if not first:
                    wait_scatter(1 - j)
                issue_gather(1 - j)
                issue_col(ci + 1, 1 - j)

            wait_gather(j)
            scale_rows(rowsb[j], ewb[j], None)
            wait_col(ci, j)
            start_scatter(j)

            @pl.when(ci + 2 < CHUNKS)
            def _():
                issue_rowew(ci + 2, j)

        def first_pair(i, carry):
            step(0, 0, True)
            step(1, 1, False)
            return carry

        def pair(i, carry):
            ci = 2 * i
            step(ci, 0, False)
            step(ci + 1, 1, False)
            return carry

        first_pair(0, 0)
        lax.fori_loop(1, CHUNKS // 2, pair, 0)
        wait_scatter(0)
        wait_scatter(1)
        plsc.subcore_barrier()
        pltpu.sync_copy(acc.at[pl.ds(s * RPT, RPT)],
                        out_hbm.at[c, pl.ds(s * RPT, RPT)])

    return pl.kernel(
        body,
        mesh=_mesh(),
        out_type=jax.ShapeDtypeStruct((NC, N_PAD, d), jnp.float32),
        scratch_types=[
            pltpu.VMEM((K,), jnp.int32),
            pltpu.VMEM((K,), jnp.int32),
            pltpu.VMEM((K,), jnp.int32),
            pltpu.VMEM((K,), jnp.int32),
            pltpu.VMEM((K,), jnp.float32),
            pltpu.VMEM((K,), jnp.float32),
            pltpu.VMEM((K, d), jnp.float32),
            pltpu.VMEM((K, d), jnp.float32),
            pltpu.VMEM_SHARED((N_PAD, d), jnp.float32),
            pltpu.SemaphoreType.DMA,
            pltpu.SemaphoreType.DMA,
            pltpu.SemaphoreType.DMA,
            pltpu.SemaphoreType.DMA,
            pltpu.SemaphoreType.DMA,
            pltpu.SemaphoreType.DMA,
            pltpu.SemaphoreType.DMA,
            pltpu.SemaphoreType.DMA,
        ],
        **_SC_PARAMS,
    )


_BR = 1024
_GRID = (N_PAD // _BR,)


def _tc_first(x_ref, w_ref, dinvb_ref, o_ref):
    o_ref[...] = dinvb_ref[...] * jnp.dot(
        x_ref[...], w_ref[...], preferred_element_type=jnp.float32)


def _tc_mid(acc_ref, g_ref, dinvb_ref, w_ref, b_ref, o_ref):
    dinvb = dinvb_ref[...]
    z = dinvb[:, : g_ref.shape[1]] * (acc_ref[0] + acc_ref[1] + g_ref[...]) \
        + b_ref[...][0:1, :]
    h = jnp.maximum(z, 0.0)
    o_ref[...] = dinvb[:, : w_ref.shape[1]] * jnp.dot(
        h, w_ref[...], preferred_element_type=jnp.float32)


def _tc_last(acc_ref, g_ref, dinvb_ref, b_ref, o_ref):
    o_ref[...] = dinvb_ref[...][:, :2] * (
        acc_ref[0][:, :2] + acc_ref[1][:, :2] + g_ref[...][:, :2]
    ) + b_ref[...][0:1, :]


def _row_spec(d):
    return pl.BlockSpec((_BR, d), lambda i: (i, 0))


def _acc_spec(d):
    return pl.BlockSpec((NC, _BR, d), lambda i: (0, i, 0))


def _full_spec(a, b):
    return pl.BlockSpec((a, b), lambda i: (0, 0))


def _tc_first_call(xp, W1, dinvb):
    return pl.pallas_call(
        _tc_first,
        grid=_GRID,
        in_specs=[_row_spec(128), _full_spec(128, 128), _row_spec(128)],
        out_specs=_row_spec(128),
        out_shape=jax.ShapeDtypeStruct((N_PAD, 128), jnp.float32),
    )(xp, W1, dinvb)


def _tc_mid_call(acc, g, dinvb, W, b2d, dout):
    din = g.shape[1]
    return pl.pallas_call(
        _tc_mid,
        grid=_GRID,
        in_specs=[_acc_spec(din), _row_spec(din), _row_spec(128),
                  _full_spec(128, dout), _full_spec(8, din)],
        out_specs=_row_spec(dout),
        out_shape=jax.ShapeDtypeStruct((N_PAD, dout), jnp.float32),
    )(acc, g, dinvb, W, b2d)


def _tc_last_call(acc, g, dinvb, b2d):
    return pl.pallas_call(
        _tc_last,
        grid=_GRID,
        in_specs=[_acc_spec(128), _row_spec(128), _row_spec(128),
                  _full_spec(8, 2)],
        out_specs=_row_spec(2),
        out_shape=jax.ShapeDtypeStruct((N_PAD, 2), jnp.float32),
    )(acc, g, dinvb, b2d)


def kernel(x, edge_index, edge_attr, W1, b1, W2, b2, W3, b3):
    row = edge_index[0].astype(jnp.int32)
    col = edge_index[1].astype(jnp.int32)
    ew = edge_attr.astype(jnp.float32)

    pe = E_PAD - N_EDGES
    pad_idx = jnp.arange(pe, dtype=jnp.int32) % N_PAD
    rowp = jnp.concatenate([row, pad_idx]).reshape(NW, CHUNKS, K)
    colp = jnp.concatenate([col, pad_idx]).reshape(NW, CHUNKS, K)
    ewp = jnp.pad(ew, (0, pe)).reshape(NW, CHUNKS, K)
    xp = jnp.pad(x, ((0, N_PAD - N_NODES), (0, 0)))

    zero1 = jnp.zeros((N_PAD,), jnp.float32)
    zero128 = jnp.zeros((N_PAD, 128), jnp.float32)

    deg_parts = _deg_kernel()(ewp, colp, zero1)
    deg = deg_parts[0] + deg_parts[1] + 1.0
    dinv = jnp.where(deg > 0, lax.rsqrt(deg), 0.0)
    dinvb = jnp.broadcast_to(dinv[:, None], (N_PAD, 128))

    b1b = jnp.broadcast_to(b1[None, :], (8, 128))
    b2b = jnp.broadcast_to(b2[None, :], (8, 128))
    b3b = jnp.broadcast_to(b3[None, :], (8, 2))
    W3p = jnp.zeros((128, 128), jnp.float32).at[:, :2].set(W3)

    g1 = _tc_first_call(xp, W1, dinvb)
    acc1 = _make_prop(128)(g1, rowp, colp, ewp, zero128)
    g2 = _tc_mid_call(acc1, g1, dinvb, W2, b1b, 128)
    acc2 = _make_prop(128)(g2, rowp, colp, ewp, zero128)
    g3 = _tc_mid_call(acc2, g2, dinvb, W3p, b2b, 128)
    acc3 = _make_prop(128)(g3, rowp, colp, ewp, zero128)
    out = _tc_last_call(acc3, g3, dinvb, b3b)
    return out[:N_NODES]


# speedup vs baseline: 1.9266x; 1.3370x over previous
import functools

import jax
import jax.numpy as jnp
from jax import lax
from jax.experimental import pallas as pl
from jax.experimental.pallas import tpu as pltpu
from jax.experimental.pallas import tpu_sc as plsc

N_NODES = 10000
N_EDGES = 320000
NC = 2
NS = 16
NW = NC * NS
N_PAD = 10240
RPT = N_PAD // NS
K = 64
CHUNKS = 158
EPW = CHUNKS * K
E_PAD = EPW * NW

_SC_PARAMS = dict(
    compiler_params=pltpu.CompilerParams(needs_layout_passes=False),
)


def _mesh():
    return plsc.VectorSubcoreMesh(core_axis_name="c", subcore_axis_name="s")


@functools.lru_cache(maxsize=None)
def _deg_kernel():
    def body(ew_hbm, col_hbm, zero_hbm, out_hbm, colv, ewv, acc, sem):
        c = lax.axis_index("c")
        s = lax.axis_index("s")
        w = c * NS + s
        pltpu.sync_copy(zero_hbm.at[pl.ds(s * RPT, RPT)],
                        acc.at[pl.ds(s * RPT, RPT)])
        pltpu.sync_copy(col_hbm.at[w], colv)
        pltpu.sync_copy(ew_hbm.at[w], ewv)
        plsc.subcore_barrier()

        def chunk(i, carry):
            pltpu.sync_copy(ewv.at[i], acc.at[colv.at[i]], add=True)
            return carry

        lax.fori_loop(0, CHUNKS, chunk, 0)
        plsc.subcore_barrier()
        pltpu.sync_copy(acc.at[pl.ds(s * RPT, RPT)],
                        out_hbm.at[c, pl.ds(s * RPT, RPT)])

    return pl.kernel(
        body,
        mesh=_mesh(),
        out_type=jax.ShapeDtypeStruct((NC, N_PAD), jnp.float32),
        scratch_types=[
            pltpu.VMEM((CHUNKS, K), jnp.int32),
            pltpu.VMEM((CHUNKS, K), jnp.float32),
            pltpu.VMEM_SHARED((N_PAD,), jnp.float32),
            pltpu.SemaphoreType.DMA,
        ],
        **_SC_PARAMS,
    )


@functools.lru_cache(maxsize=None)
def _make_prop(d):
    def scale_rows(rows, ewv, _):
        for j16 in range(K // 16):
            ew16 = ewv[pl.ds(j16 * 16, 16)]
            for jj in range(16):
                j = j16 * 16 + jj
                wb = ew16.at[jnp.full((16,), jj, jnp.int32)].get(
                    mode="promise_in_bounds")
                for cc in range(d // 16):
                    rows[j, pl.ds(cc * 16, 16)] = (
                        rows[j, pl.ds(cc * 16, 16)] * wb)

    def body(g_hbm, row_hbm, col_hbm, ew_hbm, zero_hbm, out_hbm,
             row0, row1, col0, col1, ew0, ew1, rows0, rows1, acc,
             gsem0, gsem1, isem0, isem1, csem0, csem1, ssem0, ssem1):
        c = lax.axis_index("c")
        s = lax.axis_index("s")
        w = c * NS + s
        pltpu.sync_copy(zero_hbm.at[pl.ds(s * RPT, RPT)],
                        acc.at[pl.ds(s * RPT, RPT)])
        plsc.subcore_barrier()

        rowb = (row0, row1)
        colb = (col0, col1)
        ewb = (ew0, ew1)
        rowsb = (rows0, rows1)
        gsem = (gsem0, gsem1)
        isem = (isem0, isem1)
        csem = (csem0, csem1)
        ssem = (ssem0, ssem1)

        def issue_rowew(ci, j):
            pltpu.async_copy(row_hbm.at[w, ci], rowb[j], isem[j])
            pltpu.async_copy(ew_hbm.at[w, ci], ewb[j], isem[j])

        def wait_rowew(ci, j):
            pltpu.make_async_copy(row_hbm.at[w, ci], rowb[j], isem[j]).wait()
            pltpu.make_async_copy(ew_hbm.at[w, ci], ewb[j], isem[j]).wait()

        def issue_col(ci, j):
            pltpu.async_copy(col_hbm.at[w, ci], colb[j], csem[j])

        def wait_col(ci, j):
            pltpu.make_async_copy(col_hbm.at[w, ci], colb[j], csem[j]).wait()

        def issue_gather(j):
            pltpu.async_copy(g_hbm.at[rowb[j]], rowsb[j], gsem[j])

        def wait_gather(j):
            pltpu.make_async_copy(g_hbm.at[rowb[j]], rowsb[j], gsem[j]).wait()

        def start_scatter(j):
            pltpu.async_copy(rowsb[j], acc.at[colb[j]], ssem[j], add=True)

        def wait_scatter(j):
            pltpu.make_async_copy(rowsb[j], acc.at[colb[j]], ssem[j]).wait()

        issue_rowew(0, 0)
        wait_rowew(0, 0)
        issue_gather(0)
        issue_col(0, 0)
        issue_rowew(1, 1)

        def step(ci, j, first):
            @pl.when(ci + 1 < CHUNKS)
            def _():
                wait_rowew(ci + 1, 1 - j)
                if not first:
                    wait_scatter(1 - j)
                issue_gather(1 - j)
                issue_col(ci + 1, 1 - j)

            wait_gather(j)
            scale_rows(rowsb[j], ewb[j], None)
            wait_col(ci, j)
            start_scatter(j)

            @pl.when(ci + 2 < CHUNKS)
            def _():
                issue_rowew(ci + 2, j)

        def first_pair(i, carry):
            step(0, 0, True)
            step(1, 1, False)
            return carry

        def pair(i, carry):
            ci = 2 * i
            step(ci, 0, False)
            step(ci + 1, 1, False)
            return carry

        first_pair(0, 0)
        lax.fori_loop(1, CHUNKS // 2, pair, 0)
        wait_scatter(0)
        wait_scatter(1)
        plsc.subcore_barrier()
        pltpu.sync_copy(acc.at[pl.ds(s * RPT, RPT)],
                        out_hbm.at[c, pl.ds(s * RPT, RPT)])

    return pl.kernel(
        body,
        mesh=_mesh(),
        out_type=jax.ShapeDtypeStruct((NC, N_PAD, d), jnp.float32),
        scratch_types=[
            pltpu.VMEM((K,), jnp.int32),
            pltpu.VMEM((K,), jnp.int32),
            pltpu.VMEM((K,), jnp.int32),
            pltpu.VMEM((K,), jnp.int32),
            pltpu.VMEM((K,), jnp.float32),
            pltpu.VMEM((K,), jnp.float32),
            pltpu.VMEM((K, d), jnp.float32),
            pltpu.VMEM((K, d), jnp.float32),
            pltpu.VMEM_SHARED((N_PAD, d), jnp.float32),
            pltpu.SemaphoreType.DMA,
            pltpu.SemaphoreType.DMA,
            pltpu.SemaphoreType.DMA,
            pltpu.SemaphoreType.DMA,
            pltpu.SemaphoreType.DMA,
            pltpu.SemaphoreType.DMA,
            pltpu.SemaphoreType.DMA,
            pltpu.SemaphoreType.DMA,
        ],
        **_SC_PARAMS,
    )


KL = 128
CHL = EPW // KL


@functools.lru_cache(maxsize=None)
def _prop_lite_kernel():
    def body(g3t_hbm, row_hbm, col_hbm, ew_hbm, out_hbm,
             rowv, colv, ewv, m0v, m1v, g3av, g3bv, zbuf,
             acc0, acc1, ssem, dsem):
        c = lax.axis_index("c")
        s = lax.axis_index("s")
        w = c * NS + s
        for q in range(RPT // 16):
            zbuf[pl.ds(q * 16, 16)] = jnp.zeros((16,), jnp.float32)
        pltpu.sync_copy(zbuf, acc0.at[pl.ds(s * RPT, RPT)])
        pltpu.sync_copy(zbuf, acc1.at[pl.ds(s * RPT, RPT)])
        pltpu.sync_copy(g3t_hbm.at[0], g3av)
        pltpu.sync_copy(g3t_hbm.at[1], g3bv)
        pltpu.sync_copy(row_hbm.at[w], rowv)
        pltpu.sync_copy(col_hbm.at[w], colv)
        pltpu.sync_copy(ew_hbm.at[w], ewv)
        plsc.subcore_barrier()

        def grp(i, carry):
            for q in range(KL // 16):
                sl = pl.ds(q * 16, 16)
                r16 = rowv[i, sl]
                w16 = ewv[i, sl]
                m0v[i, sl] = plsc.load_gather(g3av, [r16]) * w16
                m1v[i, sl] = plsc.load_gather(g3bv, [r16]) * w16
            pltpu.async_copy(m0v.at[i], acc0.at[colv.at[i]], ssem, add=True)
            pltpu.async_copy(m1v.at[i], acc1.at[colv.at[i]], ssem, add=True)
            return carry

        lax.fori_loop(0, CHL, grp, 0)

        def drain(i, carry):
            pltpu.make_async_copy(m0v.at[i], acc0.at[colv.at[i]], ssem).wait()
            pltpu.make_async_copy(m1v.at[i], acc1.at[colv.at[i]], ssem).wait()
            return carry

        lax.fori_loop(0, CHL, drain, 0)
        plsc.subcore_barrier()
        pltpu.sync_copy(acc0.at[pl.ds(s * RPT, RPT)],
                        out_hbm.at[c, 0, pl.ds(s * RPT, RPT)])
        pltpu.sync_copy(acc1.at[pl.ds(s * RPT, RPT)],
                        out_hbm.at[c, 1, pl.ds(s * RPT, RPT)])

    return pl.kernel(
        body,
        mesh=_mesh(),
        out_type=jax.ShapeDtypeStruct((NC, 2, N_PAD), jnp.float32),
        scratch_types=[
            pltpu.VMEM((CHL, KL), jnp.int32),
            pltpu.VMEM((CHL, KL), jnp.int32),
            pltpu.VMEM((CHL, KL), jnp.float32),
            pltpu.VMEM((CHL, KL), jnp.float32),
            pltpu.VMEM((CHL, KL), jnp.float32),
            pltpu.VMEM((N_PAD,), jnp.float32),
            pltpu.VMEM((N_PAD,), jnp.float32),
            pltpu.VMEM((RPT,), jnp.float32),
            pltpu.VMEM_SHARED((N_PAD,), jnp.float32),
            pltpu.VMEM_SHARED((N_PAD,), jnp.float32),
            pltpu.SemaphoreType.DMA,
            pltpu.SemaphoreType.DMA,
        ],
        **_SC_PARAMS,
    )


_BR = 1024
_GRID = (N_PAD // _BR,)


def _tc_first(x_ref, w_ref, dinvb_ref, o_ref):
    o_ref[...] = dinvb_ref[...] * jnp.dot(
        x_ref[...], w_ref[...], preferred_element_type=jnp.float32)


def _tc_mid(acc_ref, g_ref, dinvb_ref, w_ref, b_ref, o_ref):
    dinvb = dinvb_ref[...]
    z = dinvb[:, : g_ref.shape[1]] * (acc_ref[0] + acc_ref[1] + g_ref[...]) \
        + b_ref[...][0:1, :]
    h = jnp.maximum(z, 0.0)
    o_ref[...] = dinvb[:, : w_ref.shape[1]] * jnp.dot(
        h, w_ref[...], preferred_element_type=jnp.float32)


def _tc_mid_t(acc_ref, g_ref, dinvb_ref, w_ref, b_ref, dinvr_ref, o_ref):
    dinvb = dinvb_ref[...]
    z = dinvb * (acc_ref[0] + acc_ref[1] + g_ref[...]) + b_ref[...][0:1, :]
    h = jnp.maximum(z, 0.0)
    g3t = lax.dot_general(w_ref[...], h, (((1,), (1,)), ((), ())),
                          preferred_element_type=jnp.float32)
    o_ref[...] = g3t * dinvr_ref[...][0:1, :]


def _tc_last_t(acc_ref, g_ref, dinvr_ref, b_ref, o_ref):
    o_ref[...] = dinvr_ref[...][0:1, :] * (
        acc_ref[0] + acc_ref[1] + g_ref[...]
    ) + b_ref[...][:, 0:1]


def _row_spec(d):
    return pl.BlockSpec((_BR, d), lambda i: (i, 0))


def _acc_spec(d):
    return pl.BlockSpec((NC, _BR, d), lambda i: (0, i, 0))


def _full_spec(a, b):
    return pl.BlockSpec((a, b), lambda i: (0, 0))


def _tc_first_call(xp, W1, dinvb):
    return pl.pallas_call(
        _tc_first,
        grid=_GRID,
        in_specs=[_row_spec(128), _full_spec(128, 128), _row_spec(128)],
        out_specs=_row_spec(128),
        out_shape=jax.ShapeDtypeStruct((N_PAD, 128), jnp.float32),
    )(xp, W1, dinvb)


def _tc_mid_call(acc, g, dinvb, W, b2d, dout):
    din = g.shape[1]
    return pl.pallas_call(
        _tc_mid,
        grid=_GRID,
        in_specs=[_acc_spec(din), _row_spec(din), _row_spec(128),
                  _full_spec(128, dout), _full_spec(8, din)],
        out_specs=_row_spec(dout),
        out_shape=jax.ShapeDtypeStruct((N_PAD, dout), jnp.float32),
    )(acc, g, dinvb, W, b2d)


def _col_spec(d):
    return pl.BlockSpec((d, _BR), lambda i: (0, i))


def _tc_mid_t_call(acc, g, dinvb, W3T, b2d, dinvr):
    return pl.pallas_call(
        _tc_mid_t,
        grid=_GRID,
        in_specs=[_acc_spec(128), _row_spec(128), _row_spec(128),
                  _full_spec(2, 128), _full_spec(8, 128), _col_spec(8)],
        out_specs=_col_spec(2),
        out_shape=jax.ShapeDtypeStruct((2, N_PAD), jnp.float32),
    )(acc, g, dinvb, W3T, b2d, dinvr)


def _tc_last_t_call(acc, g, dinvr, b3c):
    return pl.pallas_call(
        _tc_last_t,
        grid=_GRID,
        in_specs=[pl.BlockSpec((NC, 2, _BR), lambda i: (0, 0, i)),
                  _col_spec(2), _col_spec(8), _full_spec(2, 128)],
        out_specs=_col_spec(2),
        out_shape=jax.ShapeDtypeStruct((2, N_PAD), jnp.float32),
    )(acc, g, dinvr, b3c)


def kernel(x, edge_index, edge_attr, W1, b1, W2, b2, W3, b3):
    row = edge_index[0].astype(jnp.int32)
    col = edge_index[1].astype(jnp.int32)
    ew = edge_attr.astype(jnp.float32)

    pe = E_PAD - N_EDGES
    pad_idx = jnp.arange(pe, dtype=jnp.int32) % N_PAD
    rowp = jnp.concatenate([row, pad_idx]).reshape(NW, CHUNKS, K)
    colp = jnp.concatenate([col, pad_idx]).reshape(NW, CHUNKS, K)
    ewp = jnp.pad(ew, (0, pe)).reshape(NW, CHUNKS, K)
    xp = jnp.pad(x, ((0, N_PAD - N_NODES), (0, 0)))

    zero1 = jnp.zeros((N_PAD,), jnp.float32)
    zero128 = jnp.zeros((N_PAD, 128), jnp.float32)

    deg_parts = _deg_kernel()(ewp, colp, zero1)
    deg = deg_parts[0] + deg_parts[1] + 1.0
    dinv = jnp.where(deg > 0, lax.rsqrt(deg), 0.0)
    dinvb = jnp.broadcast_to(dinv[:, None], (N_PAD, 128))

    b1b = jnp.broadcast_to(b1[None, :], (8, 128))
    b2b = jnp.broadcast_to(b2[None, :], (8, 128))
    b3c = jnp.broadcast_to(b3[:, None], (2, 128))
    W3T = W3.T
    dinvr = jnp.broadcast_to(dinv[None, :], (8, N_PAD))

    rowl = rowp.reshape(NW, CHL, KL)
    coll = colp.reshape(NW, CHL, KL)
    ewl = ewp.reshape(NW, CHL, KL)

    g1 = _tc_first_call(xp, W1, dinvb)
    acc1 = _make_prop(128)(g1, rowp, colp, ewp, zero128)
    g2 = _tc_mid_call(acc1, g1, dinvb, W2, b1b, 128)
    acc2 = _make_prop(128)(g2, rowp, colp, ewp, zero128)
    g3t = _tc_mid_t_call(acc2, g2, dinvb, W3T, b2b, dinvr)
    acc3t = _prop_lite_kernel()(g3t, rowl, coll, ewl)
    outt = _tc_last_t_call(acc3t, g3t, dinvr, b3c)
    return outt.T[:N_NODES]


# speedup vs baseline: 1.9319x; 1.0027x over previous
import functools

import jax
import jax.numpy as jnp
from jax import lax
from jax.experimental import pallas as pl
from jax.experimental.pallas import tpu as pltpu
from jax.experimental.pallas import tpu_sc as plsc

N_NODES = 10000
N_EDGES = 320000
NC = 2
NS = 16
NW = NC * NS
N_PAD = 10240
RPT = N_PAD // NS
K = 64
CHUNKS = 158
EPW = CHUNKS * K
E_PAD = EPW * NW

_SC_PARAMS = dict(
    compiler_params=pltpu.CompilerParams(needs_layout_passes=False),
)


def _mesh():
    return plsc.VectorSubcoreMesh(core_axis_name="c", subcore_axis_name="s")


@functools.lru_cache(maxsize=None)
def _deg_kernel():
    def body(ew_hbm, col_hbm, zero_hbm, out_hbm, colv, ewv, acc, sem):
        c = lax.axis_index("c")
        s = lax.axis_index("s")
        w = c * NS + s
        pltpu.sync_copy(zero_hbm.at[pl.ds(s * RPT, RPT)],
                        acc.at[pl.ds(s * RPT, RPT)])
        pltpu.sync_copy(col_hbm.at[w], colv)
        pltpu.sync_copy(ew_hbm.at[w], ewv)
        plsc.subcore_barrier()

        def chunk(i, carry):
            pltpu.sync_copy(ewv.at[i], acc.at[colv.at[i]], add=True)
            return carry

        lax.fori_loop(0, CHUNKS, chunk, 0)
        plsc.subcore_barrier()
        pltpu.sync_copy(acc.at[pl.ds(s * RPT, RPT)],
                        out_hbm.at[c, pl.ds(s * RPT, RPT)])

    return pl.kernel(
        body,
        mesh=_mesh(),
        out_type=jax.ShapeDtypeStruct((NC, N_PAD), jnp.float32),
        scratch_types=[
            pltpu.VMEM((CHUNKS, K), jnp.int32),
            pltpu.VMEM((CHUNKS, K), jnp.float32),
            pltpu.VMEM_SHARED((N_PAD,), jnp.float32),
            pltpu.SemaphoreType.DMA,
        ],
        **_SC_PARAMS,
    )


@functools.lru_cache(maxsize=None)
def _make_prop(d):
    def scale_rows(rows, eb):
        for j16 in range(K // 16):
            ew16 = plsc.bitcast(eb[1, pl.ds(j16 * 16, 16)], jnp.float32)
            for jj in range(16):
                j = j16 * 16 + jj
                wb = ew16.at[jnp.full((16,), jj, jnp.int32)].get(
                    mode="promise_in_bounds")
                for cc in range(d // 16):
                    rows[j, pl.ds(cc * 16, 16)] = (
                        rows[j, pl.ds(cc * 16, 16)] * wb)

    def body(g_hbm, e_hbm, col_hbm, out_hbm,
             eb0, eb1, col0, col1, rows0, rows1, zbuf, acc,
             gsem0, gsem1, isem0, isem1, csem0, csem1, ssem0, ssem1):
        c = lax.axis_index("c")
        s = lax.axis_index("s")
        w = c * NS + s
        for q in range(64 * d // 16):
            zbuf[q // (d // 16), pl.ds((q % (d // 16)) * 16, 16)] = (
                jnp.zeros((16,), jnp.float32))
        for t in range(RPT // 64):
            pltpu.sync_copy(zbuf, acc.at[pl.ds(s * RPT + t * 64, 64)])
        plsc.subcore_barrier()

        eb = (eb0, eb1)
        colb = (col0, col1)
        rowsb = (rows0, rows1)
        gsem = (gsem0, gsem1)
        isem = (isem0, isem1)
        csem = (csem0, csem1)
        ssem = (ssem0, ssem1)

        def issue_rowew(ci, j):
            pltpu.async_copy(e_hbm.at[w, ci], eb[j], isem[j])

        def wait_rowew(ci, j):
            pltpu.make_async_copy(e_hbm.at[w, ci], eb[j], isem[j]).wait()

        def issue_col(ci, j):
            pltpu.async_copy(col_hbm.at[w, ci], colb[j], csem[j])

        def wait_col(ci, j):
            pltpu.make_async_copy(col_hbm.at[w, ci], colb[j], csem[j]).wait()

        def issue_gather(j):
            pltpu.async_copy(g_hbm.at[eb[j].at[0]], rowsb[j], gsem[j])

        def wait_gather(j):
            pltpu.make_async_copy(g_hbm.at[eb[j].at[0]], rowsb[j],
                                  gsem[j]).wait()

        def start_scatter(j):
            pltpu.async_copy(rowsb[j], acc.at[colb[j]], ssem[j], add=True)

        def wait_scatter(j):
            pltpu.make_async_copy(rowsb[j], acc.at[colb[j]], ssem[j]).wait()

        issue_rowew(0, 0)
        wait_rowew(0, 0)
        issue_gather(0)
        issue_col(0, 0)
        issue_rowew(1, 1)

        def step(ci, j, first):
            @pl.when(ci + 1 < CHUNKS)
            def _():
                wait_rowew(ci + 1, 1 - j)
                if not first:
                    wait_scatter(1 - j)
                issue_gather(1 - j)
                issue_col(ci + 1, 1 - j)

            wait_gather(j)
            scale_rows(rowsb[j], eb[j])
            wait_col(ci, j)
            start_scatter(j)

            @pl.when(ci + 2 < CHUNKS)
            def _():
                issue_rowew(ci + 2, j)

        def first_pair(i, carry):
            step(0, 0, True)
            step(1, 1, False)
            return carry

        def pair(i, carry):
            ci = 2 * i
            step(ci, 0, False)
            step(ci + 1, 1, False)
            return carry

        first_pair(0, 0)
        lax.fori_loop(1, CHUNKS // 2, pair, 0)
        wait_scatter(0)
        wait_scatter(1)
        plsc.subcore_barrier()
        pltpu.sync_copy(acc.at[pl.ds(s * RPT, RPT)],
                        out_hbm.at[c, pl.ds(s * RPT, RPT)])

    return pl.kernel(
        body,
        mesh=_mesh(),
        out_type=jax.ShapeDtypeStruct((NC, N_PAD, d), jnp.float32),
        scratch_types=[
            pltpu.VMEM((2, K), jnp.int32),
            pltpu.VMEM((2, K), jnp.int32),
            pltpu.VMEM((K,), jnp.int32),
            pltpu.VMEM((K,), jnp.int32),
            pltpu.VMEM((K, d), jnp.float32),
            pltpu.VMEM((K, d), jnp.float32),
            pltpu.VMEM((64, d), jnp.float32),
            pltpu.VMEM_SHARED((N_PAD, d), jnp.float32),
            pltpu.SemaphoreType.DMA,
            pltpu.SemaphoreType.DMA,
            pltpu.SemaphoreType.DMA,
            pltpu.SemaphoreType.DMA,
            pltpu.SemaphoreType.DMA,
            pltpu.SemaphoreType.DMA,
            pltpu.SemaphoreType.DMA,
            pltpu.SemaphoreType.DMA,
        ],
        **_SC_PARAMS,
    )


KL = 128
CHL = EPW // KL


@functools.lru_cache(maxsize=None)
def _prop_lite_kernel():
    def body(g3t_hbm, row_hbm, col_hbm, ew_hbm, out_hbm,
             rowv, colv, ewv, m0v, m1v, g3av, g3bv, zbuf,
             acc0, acc1, ssem, dsem):
        c = lax.axis_index("c")
        s = lax.axis_index("s")
        w = c * NS + s
        for q in range(RPT // 16):
            zbuf[pl.ds(q * 16, 16)] = jnp.zeros((16,), jnp.float32)
        pltpu.sync_copy(zbuf, acc0.at[pl.ds(s * RPT, RPT)])
        pltpu.sync_copy(zbuf, acc1.at[pl.ds(s * RPT, RPT)])
        pltpu.sync_copy(g3t_hbm.at[0], g3av)
        pltpu.sync_copy(g3t_hbm.at[1], g3bv)
        pltpu.sync_copy(row_hbm.at[w], rowv)
        pltpu.sync_copy(col_hbm.at[w], colv)
        pltpu.sync_copy(ew_hbm.at[w], ewv)
        plsc.subcore_barrier()

        def grp(i, carry):
            for q in range(KL // 16):
                sl = pl.ds(q * 16, 16)
                r16 = rowv[i, sl]
                w16 = ewv[i, sl]
                m0v[i, sl] = plsc.load_gather(g3av, [r16]) * w16
                m1v[i, sl] = plsc.load_gather(g3bv, [r16]) * w16
            pltpu.async_copy(m0v.at[i], acc0.at[colv.at[i]], ssem, add=True)
            pltpu.async_copy(m1v.at[i], acc1.at[colv.at[i]], ssem, add=True)
            return carry

        lax.fori_loop(0, CHL, grp, 0)

        def drain(i, carry):
            pltpu.make_async_copy(m0v.at[i], acc0.at[colv.at[i]], ssem).wait()
            pltpu.make_async_copy(m1v.at[i], acc1.at[colv.at[i]], ssem).wait()
            return carry

        lax.fori_loop(0, CHL, drain, 0)
        plsc.subcore_barrier()
        pltpu.sync_copy(acc0.at[pl.ds(s * RPT, RPT)],
                        out_hbm.at[c, 0, pl.ds(s * RPT, RPT)])
        pltpu.sync_copy(acc1.at[pl.ds(s * RPT, RPT)],
                        out_hbm.at[c, 1, pl.ds(s * RPT, RPT)])

    return pl.kernel(
        body,
        mesh=_mesh(),
        out_type=jax.ShapeDtypeStruct((NC, 2, N_PAD), jnp.float32),
        scratch_types=[
            pltpu.VMEM((CHL, KL), jnp.int32),
            pltpu.VMEM((CHL, KL), jnp.int32),
            pltpu.VMEM((CHL, KL), jnp.float32),
            pltpu.VMEM((CHL, KL), jnp.float32),
            pltpu.VMEM((CHL, KL), jnp.float32),
            pltpu.VMEM((N_PAD,), jnp.float32),
            pltpu.VMEM((N_PAD,), jnp.float32),
            pltpu.VMEM((RPT,), jnp.float32),
            pltpu.VMEM_SHARED((N_PAD,), jnp.float32),
            pltpu.VMEM_SHARED((N_PAD,), jnp.float32),
            pltpu.SemaphoreType.DMA,
            pltpu.SemaphoreType.DMA,
        ],
        **_SC_PARAMS,
    )


_BR = 1024
_GRID = (N_PAD // _BR,)


def _tc_first(x_ref, w_ref, dinvb_ref, o_ref):
    o_ref[...] = dinvb_ref[...] * jnp.dot(
        x_ref[...], w_ref[...], preferred_element_type=jnp.float32)


def _tc_mid(acc_ref, g_ref, dinvb_ref, w_ref, b_ref, o_ref):
    dinvb = dinvb_ref[...]
    z = dinvb[:, : g_ref.shape[1]] * (acc_ref[0] + acc_ref[1] + g_ref[...]) \
        + b_ref[...][0:1, :]
    h = jnp.maximum(z, 0.0)
    o_ref[...] = dinvb[:, : w_ref.shape[1]] * jnp.dot(
        h, w_ref[...], preferred_element_type=jnp.float32)


def _tc_mid_t(acc_ref, g_ref, dinvb_ref, w_ref, b_ref, dinvr_ref, o_ref):
    dinvb = dinvb_ref[...]
    z = dinvb * (acc_ref[0] + acc_ref[1] + g_ref[...]) + b_ref[...][0:1, :]
    h = jnp.maximum(z, 0.0)
    g3t = lax.dot_general(w_ref[...], h, (((1,), (1,)), ((), ())),
                          preferred_element_type=jnp.float32)
    o_ref[...] = g3t * dinvr_ref[...][0:1, :]


def _tc_last_t(acc_ref, g_ref, dinvr_ref, b_ref, o_ref):
    o_ref[...] = dinvr_ref[...][0:1, :] * (
        acc_ref[0] + acc_ref[1] + g_ref[...]
    ) + b_ref[...][:, 0:1]


def _row_spec(d):
    return pl.BlockSpec((_BR, d), lambda i: (i, 0))


def _acc_spec(d):
    return pl.BlockSpec((NC, _BR, d), lambda i: (0, i, 0))


def _full_spec(a, b):
    return pl.BlockSpec((a, b), lambda i: (0, 0))


def _tc_first_call(xp, W1, dinvb):
    return pl.pallas_call(
        _tc_first,
        grid=_GRID,
        in_specs=[_row_spec(128), _full_spec(128, 128), _row_spec(128)],
        out_specs=_row_spec(128),
        out_shape=jax.ShapeDtypeStruct((N_PAD, 128), jnp.float32),
    )(xp, W1, dinvb)


def _tc_mid_call(acc, g, dinvb, W, b2d, dout):
    din = g.shape[1]
    return pl.pallas_call(
        _tc_mid,
        grid=_GRID,
        in_specs=[_acc_spec(din), _row_spec(din), _row_spec(128),
                  _full_spec(128, dout), _full_spec(8, din)],
        out_specs=_row_spec(dout),
        out_shape=jax.ShapeDtypeStruct((N_PAD, dout), jnp.float32),
    )(acc, g, dinvb, W, b2d)


def _col_spec(d):
    return pl.BlockSpec((d, _BR), lambda i: (0, i))


def _tc_mid_t_call(acc, g, dinvb, W3T, b2d, dinvr):
    return pl.pallas_call(
        _tc_mid_t,
        grid=_GRID,
        in_specs=[_acc_spec(128), _row_spec(128), _row_spec(128),
                  _full_spec(2, 128), _full_spec(8, 128), _col_spec(8)],
        out_specs=_col_spec(2),
        out_shape=jax.ShapeDtypeStruct((2, N_PAD), jnp.float32),
    )(acc, g, dinvb, W3T, b2d, dinvr)


def _tc_last_t_call(acc, g, dinvr, b3c):
    return pl.pallas_call(
        _tc_last_t,
        grid=_GRID,
        in_specs=[pl.BlockSpec((NC, 2, _BR), lambda i: (0, 0, i)),
                  _col_spec(2), _col_spec(8), _full_spec(2, 128)],
        out_specs=_col_spec(2),
        out_shape=jax.ShapeDtypeStruct((2, N_PAD), jnp.float32),
    )(acc, g, dinvr, b3c)


def kernel(x, edge_index, edge_attr, W1, b1, W2, b2, W3, b3):
    row = edge_index[0].astype(jnp.int32)
    col = edge_index[1].astype(jnp.int32)
    ew = edge_attr.astype(jnp.float32)

    pe = E_PAD - N_EDGES
    pad_idx = jnp.arange(pe, dtype=jnp.int32) % N_PAD
    rowp = jnp.concatenate([row, pad_idx]).reshape(NW, CHUNKS, K)
    colp = jnp.concatenate([col, pad_idx]).reshape(NW, CHUNKS, K)
    ewp = jnp.pad(ew, (0, pe)).reshape(NW, CHUNKS, K)
    xp = jnp.pad(x, ((0, N_PAD - N_NODES), (0, 0)))

    zero1 = jnp.zeros((N_PAD,), jnp.float32)
    epack = jnp.stack(
        [rowp, lax.bitcast_convert_type(ewp, jnp.int32)], axis=2)

    deg_parts = _deg_kernel()(ewp, colp, zero1)
    deg = deg_parts[0] + deg_parts[1] + 1.0
    dinv = jnp.where(deg > 0, lax.rsqrt(deg), 0.0)
    dinvb = jnp.broadcast_to(dinv[:, None], (N_PAD, 128))

    b1b = jnp.broadcast_to(b1[None, :], (8, 128))
    b2b = jnp.broadcast_to(b2[None, :], (8, 128))
    b3c = jnp.broadcast_to(b3[:, None], (2, 128))
    W3T = W3.T
    dinvr = jnp.broadcast_to(dinv[None, :], (8, N_PAD))

    rowl = rowp.reshape(NW, CHL, KL)
    coll = colp.reshape(NW, CHL, KL)
    ewl = ewp.reshape(NW, CHL, KL)

    g1 = _tc_first_call(xp, W1, dinvb)
    acc1 = _make_prop(128)(g1, epack, colp)
    g2 = _tc_mid_call(acc1, g1, dinvb, W2, b1b, 128)
    acc2 = _make_prop(128)(g2, epack, colp)
    g3t = _tc_mid_t_call(acc2, g2, dinvb, W3T, b2b, dinvr)
    acc3t = _prop_lite_kernel()(g3t, rowl, coll, ewl)
    outt = _tc_last_t_call(acc3t, g3t, dinvr, b3c)
    return outt.T[:N_NODES]
